# baseline (speedup 1.0000x reference)
import functools

import jax
import jax.numpy as jnp
from jax import lax
from jax.experimental import pallas as pl
from jax.experimental.pallas import tpu as pltpu
from jax.experimental.pallas import tpu_sc as plsc

N_NODES = 100000
D_FEAT = 512
N_SEG = 1024

BLK = 2048
GRID = 49

NW = 16
CHUNK = 6256
LAST_CHUNK = N_NODES - (NW - 1) * CHUNK
NVEC = CHUNK // 16
LAST_NVEC = LAST_CHUNK // 16
SEG_PER_W = 128


def _mlp_body(nf_ref, co_ref, w1_ref, b1_ref, w2_ref, b2_ref, out_ref):
    z = jnp.dot(nf_ref[...].astype(jnp.bfloat16), w1_ref[:D_FEAT, :],
                preferred_element_type=jnp.float32)
    z += lax.dot_general(co_ref[...].astype(jnp.bfloat16), w1_ref[D_FEAT:, :],
                         (((0,), (0,)), ((), ())),
                         preferred_element_type=jnp.float32)
    h = jnp.tanh(z + b1_ref[...].reshape(1, D_FEAT))
    e = lax.dot_general(w2_ref[...], h, (((0,), (1,)), ((), ())),
                        preferred_element_type=jnp.float32)
    out_ref[...] = (e + b2_ref[0]).reshape(BLK)


def _site_energy(coordinate, node_feat, W1, b1, W2, b2):
    w1_bf = W1.astype(jnp.bfloat16)
    return pl.pallas_call(
        _mlp_body,
        grid=(GRID,),
        in_specs=[
            pl.BlockSpec((BLK, D_FEAT), lambda i: (i, 0)),
            pl.BlockSpec((3, BLK), lambda i: (0, i)),
            pl.BlockSpec((D_FEAT + 3, D_FEAT), lambda i: (0, 0)),
            pl.BlockSpec((D_FEAT,), lambda i: (0,)),
            pl.BlockSpec((D_FEAT, 1), lambda i: (0, 0)),
            pl.BlockSpec((1,), lambda i: (0,)),
        ],
        out_specs=pl.BlockSpec((BLK,), lambda i: (i,)),
        out_shape=jax.ShapeDtypeStruct((N_NODES,), jnp.float32),
    )(node_feat, coordinate.T, w1_bf, b1, W2, b2)


def _seg_body(site_hbm, batch_hbm, out_hbm, site_v, idx_v, acc_v, red_v,
              out_stage_v, shared):
    c = lax.axis_index("c")
    s = lax.axis_index("s")
    lane = lax.iota(jnp.int32, 16)

    @pl.when(c == 0)
    def _phase1():
        base = s * CHUNK
        @pl.when(s < NW - 1)
        def _full():
            pltpu.sync_copy(site_hbm.at[pl.ds(base, CHUNK)], site_v)
            pltpu.sync_copy(batch_hbm.at[pl.ds(base, CHUNK)],
                            idx_v.at[pl.ds(16, CHUNK)])

        @pl.when(s == NW - 1)
        def _tail():
            pltpu.sync_copy(site_hbm.at[pl.ds(base, LAST_CHUNK)],
                            site_v.at[pl.ds(0, LAST_CHUNK)])
            pltpu.sync_copy(batch_hbm.at[pl.ds(base, LAST_CHUNK)],
                            idx_v.at[pl.ds(16, LAST_CHUNK)])

        zeros = jnp.zeros((16,), jnp.float32)

        def zero_body(i, carry):
            acc_v[pl.ds(i * 16, 16)] = zeros
            return carry

        lax.fori_loop(0, N_SEG // 16, zero_body, 0)

        def one_vec(v):
            e = site_v[pl.ds(v * 16, 16)]
            idx = idx_v[pl.ds(16 + v * 16, 16)]
            idx_prev = idx_v[pl.ds(15 + v * 16, 16)]
            idx_next = idx_v[pl.ds(17 + v * 16, 16)]
            is_first = (lane == 0) | (idx != idx_prev)
            is_last = (lane == 15) | (idx != idx_next)
            csum = plsc.cumsum(e)
            plsc.addupdate_scatter(acc_v, [idx], csum, mask=is_last)
            plsc.addupdate_scatter(acc_v, [idx], e - csum, mask=is_first)

        def vec_body2(v, carry):
            one_vec(2 * v)
            one_vec(2 * v + 1)
            return carry

        nvec = jnp.where(s == NW - 1, LAST_NVEC, NVEC)
        lax.fori_loop(0, nvec // 2, vec_body2, 0)
        one_vec(nvec - 1)
        pltpu.sync_copy(acc_v, shared.at[s])

    plsc.subcore_barrier()

    @pl.when((c == 0) & (s < N_SEG // SEG_PER_W))
    def _phase2():
        col = s * SEG_PER_W
        pltpu.sync_copy(shared.at[:, pl.ds(col, SEG_PER_W)], red_v)
        for j in range(SEG_PER_W // 16):
            tot = jnp.zeros((16,), jnp.float32)
            for r in range(NW):
                tot = tot + red_v[r, pl.ds(j * 16, 16)]
            out_stage_v[pl.ds(j * 16, 16)] = tot
        pltpu.sync_copy(out_stage_v, out_hbm.at[pl.ds(col, SEG_PER_W)])


@functools.cache
def _make_seg_kernel():
  return pl.kernel(
    _seg_body,
    out_type=jax.ShapeDtypeStruct((N_SEG,), jnp.float32),
    mesh=plsc.VectorSubcoreMesh(core_axis_name="c", subcore_axis_name="s"),
    compiler_params=pltpu.CompilerParams(needs_layout_passes=False),
    scratch_types=[
        pltpu.VMEM((CHUNK,), jnp.float32),
        pltpu.VMEM((CHUNK + 32,), jnp.int32),
        pltpu.VMEM((N_SEG,), jnp.float32),
        pltpu.VMEM((NW, SEG_PER_W), jnp.float32),
        pltpu.VMEM((SEG_PER_W,), jnp.float32),
        pltpu.VMEM_SHARED((NW, N_SEG), jnp.float32),
    ],
  )


def kernel(coordinate, node_feat, batch, W1, b1, W2, b2):
    site = _site_energy(coordinate, node_feat, W1, b1, W2, b2)
    energy = _make_seg_kernel()(site, batch.astype(jnp.int32))
    return site, energy


# speedup vs baseline: 1.0173x; 1.0173x over previous
import functools

import jax
import jax.numpy as jnp
from jax import lax
from jax.experimental import pallas as pl
from jax.experimental.pallas import tpu as pltpu
from jax.experimental.pallas import tpu_sc as plsc

N_NODES = 100000
D_FEAT = 512
N_SEG = 1024

BLK = 2048
GRID = 49

NW = 16
CHUNK = 6256
LAST_CHUNK = N_NODES - (NW - 1) * CHUNK
NVEC = CHUNK // 16
LAST_NVEC = LAST_CHUNK // 16
SEG_PER_W = 128


def _mlp_body(nf0_ref, nf1_ref, co_ref, w1a0_ref, w1a1_ref, w1b_ref, b1_ref,
              w2t_ref, b2_ref, out_ref):
    z = jnp.dot(nf0_ref[...].astype(jnp.bfloat16), w1a0_ref[...],
                preferred_element_type=jnp.float32)
    z += jnp.dot(nf1_ref[...].astype(jnp.bfloat16), w1a1_ref[...],
                 preferred_element_type=jnp.float32)
    z += lax.dot_general(co_ref[...].astype(jnp.bfloat16), w1b_ref[...],
                         (((0,), (0,)), ((), ())),
                         preferred_element_type=jnp.float32)
    h = jnp.tanh(z + b1_ref[...])
    e = lax.dot_general(w2t_ref[...], h, (((1,), (1,)), ((), ())),
                        preferred_element_type=jnp.float32)
    out_ref[...] = (e + b2_ref[0, 0]).reshape(BLK)


def _site_energy(coordinate, node_feat, W1, b1, W2, b2):
    w1a = W1[:D_FEAT].astype(jnp.bfloat16)
    w1b = W1[D_FEAT:].astype(jnp.bfloat16)
    b1r = b1.reshape(1, -1)
    w2t = W2.reshape(-1, 1).T
    b2r = b2.reshape(1, 1)
    return pl.pallas_call(
        _mlp_body,
        grid=(GRID,),
        in_specs=[
            pl.BlockSpec((BLK, D_FEAT // 2), lambda i: (i, 0)),
            pl.BlockSpec((BLK, D_FEAT // 2), lambda i: (i, 1)),
            pl.BlockSpec((3, BLK), lambda i: (0, i)),
            pl.BlockSpec((D_FEAT // 2, D_FEAT), lambda i: (0, 0)),
            pl.BlockSpec((D_FEAT // 2, D_FEAT), lambda i: (1, 0)),
            pl.BlockSpec((3, D_FEAT), lambda i: (0, 0)),
            pl.BlockSpec((1, D_FEAT), lambda i: (0, 0)),
            pl.BlockSpec((1, D_FEAT), lambda i: (0, 0)),
            pl.BlockSpec((1, 1), lambda i: (0, 0)),
        ],
        out_specs=pl.BlockSpec((BLK,), lambda i: (i,)),
        out_shape=jax.ShapeDtypeStruct((N_NODES,), jnp.float32),
    )(node_feat, node_feat, coordinate.T, w1a, w1a, w1b, b1r, w2t, b2r)


def _seg_body(site_hbm, batch_hbm, out_hbm, site_v, idx_v, acc_v, red_v,
              out_stage_v, shared):
    c = lax.axis_index("c")
    s = lax.axis_index("s")
    lane = lax.iota(jnp.int32, 16)

    @pl.when(c == 0)
    def _phase1():
        base = s * CHUNK
        @pl.when(s < NW - 1)
        def _full():
            pltpu.sync_copy(site_hbm.at[pl.ds(base, CHUNK)], site_v)
            pltpu.sync_copy(batch_hbm.at[pl.ds(base, CHUNK)],
                            idx_v.at[pl.ds(16, CHUNK)])

        @pl.when(s == NW - 1)
        def _tail():
            pltpu.sync_copy(site_hbm.at[pl.ds(base, LAST_CHUNK)],
                            site_v.at[pl.ds(0, LAST_CHUNK)])
            pltpu.sync_copy(batch_hbm.at[pl.ds(base, LAST_CHUNK)],
                            idx_v.at[pl.ds(16, LAST_CHUNK)])

        zeros = jnp.zeros((16,), jnp.float32)

        def zero_body(i, carry):
            acc_v[pl.ds(i * 16, 16)] = zeros
            return carry

        lax.fori_loop(0, N_SEG // 16, zero_body, 0)

        def one_vec(v):
            e = site_v[pl.ds(v * 16, 16)]
            idx = idx_v[pl.ds(16 + v * 16, 16)]
            idx_prev = idx_v[pl.ds(15 + v * 16, 16)]
            idx_next = idx_v[pl.ds(17 + v * 16, 16)]
            is_first = (lane == 0) | (idx != idx_prev)
            is_last = (lane == 15) | (idx != idx_next)
            csum = plsc.cumsum(e)
            plsc.addupdate_scatter(acc_v, [idx], csum, mask=is_last)
            plsc.addupdate_scatter(acc_v, [idx], e - csum, mask=is_first)

        def vec_body2(v, carry):
            one_vec(2 * v)
            one_vec(2 * v + 1)
            return carry

        nvec = jnp.where(s == NW - 1, LAST_NVEC, NVEC)
        lax.fori_loop(0, nvec // 2, vec_body2, 0)
        one_vec(nvec - 1)
        pltpu.sync_copy(acc_v, shared.at[s])

    plsc.subcore_barrier()

    @pl.when((c == 0) & (s < N_SEG // SEG_PER_W))
    def _phase2():
        col = s * SEG_PER_W
        pltpu.sync_copy(shared.at[:, pl.ds(col, SEG_PER_W)], red_v)
        for j in range(SEG_PER_W // 16):
            tot = jnp.zeros((16,), jnp.float32)
            for r in range(NW):
                tot = tot + red_v[r, pl.ds(j * 16, 16)]
            out_stage_v[pl.ds(j * 16, 16)] = tot
        pltpu.sync_copy(out_stage_v, out_hbm.at[pl.ds(col, SEG_PER_W)])


@functools.cache
def _make_seg_kernel():
  return pl.kernel(
    _seg_body,
    out_type=jax.ShapeDtypeStruct((N_SEG,), jnp.float32),
    mesh=plsc.VectorSubcoreMesh(core_axis_name="c", subcore_axis_name="s"),
    compiler_params=pltpu.CompilerParams(needs_layout_passes=False),
    scratch_types=[
        pltpu.VMEM((CHUNK,), jnp.float32),
        pltpu.VMEM((CHUNK + 32,), jnp.int32),
        pltpu.VMEM((N_SEG,), jnp.float32),
        pltpu.VMEM((NW, SEG_PER_W), jnp.float32),
        pltpu.VMEM((SEG_PER_W,), jnp.float32),
        pltpu.VMEM_SHARED((NW, N_SEG), jnp.float32),
    ],
  )


def kernel(coordinate, node_feat, batch, W1, b1, W2, b2):
    site = _site_energy(coordinate, node_feat, W1, b1, W2, b2)
    energy = _make_seg_kernel()(site, batch.astype(jnp.int32))
    return site, energy


# speedup vs baseline: 1.0232x; 1.0058x over previous
import functools

import jax
import jax.numpy as jnp
from jax import lax
from jax.experimental import pallas as pl
from jax.experimental.pallas import tpu as pltpu
from jax.experimental.pallas import tpu_sc as plsc

N_NODES = 100000
D_FEAT = 512
N_SEG = 1024

BLK = 3072
GRID = 33

NW = 16
CHUNK = 6256
LAST_CHUNK = N_NODES - (NW - 1) * CHUNK
NVEC = CHUNK // 16
LAST_NVEC = LAST_CHUNK // 16
SEG_PER_W = 128


def _mlp_body(nf0_ref, nf1_ref, co_ref, w1a0_ref, w1a1_ref, w1b_ref, b1_ref,
              w2t_ref, b2_ref, out_ref):
    z = jnp.dot(nf0_ref[...].astype(jnp.bfloat16), w1a0_ref[...],
                preferred_element_type=jnp.float32)
    z += jnp.dot(nf1_ref[...].astype(jnp.bfloat16), w1a1_ref[...],
                 preferred_element_type=jnp.float32)
    z += lax.dot_general(co_ref[...].astype(jnp.bfloat16), w1b_ref[...],
                         (((0,), (0,)), ((), ())),
                         preferred_element_type=jnp.float32)
    h = jnp.tanh(z + b1_ref[...])
    e = lax.dot_general(w2t_ref[...], h, (((1,), (1,)), ((), ())),
                        preferred_element_type=jnp.float32)
    out_ref[...] = (e + b2_ref[0, 0]).reshape(BLK)


def _site_energy(coordinate, node_feat, W1, b1, W2, b2):
    w1a = W1[:D_FEAT].astype(jnp.bfloat16)
    w1b = W1[D_FEAT:].astype(jnp.bfloat16)
    b1r = b1.reshape(1, -1)
    w2t = W2.reshape(-1, 1).T
    b2r = b2.reshape(1, 1)
    return pl.pallas_call(
        _mlp_body,
        grid=(GRID,),
        in_specs=[
            pl.BlockSpec((BLK, D_FEAT // 2), lambda i: (i, 0)),
            pl.BlockSpec((BLK, D_FEAT // 2), lambda i: (i, 1)),
            pl.BlockSpec((3, BLK), lambda i: (0, i)),
            pl.BlockSpec((D_FEAT // 2, D_FEAT), lambda i: (0, 0)),
            pl.BlockSpec((D_FEAT // 2, D_FEAT), lambda i: (1, 0)),
            pl.BlockSpec((3, D_FEAT), lambda i: (0, 0)),
            pl.BlockSpec((1, D_FEAT), lambda i: (0, 0)),
            pl.BlockSpec((1, D_FEAT), lambda i: (0, 0)),
            pl.BlockSpec((1, 1), lambda i: (0, 0)),
        ],
        out_specs=pl.BlockSpec((BLK,), lambda i: (i,)),
        out_shape=jax.ShapeDtypeStruct((N_NODES,), jnp.float32),
    )(node_feat, node_feat, coordinate.T, w1a, w1a, w1b, b1r, w2t, b2r)


def _seg_body(site_hbm, batch_hbm, out_hbm, site_v, idx_v, acc_v, red_v,
              out_stage_v, shared):
    c = lax.axis_index("c")
    s = lax.axis_index("s")
    lane = lax.iota(jnp.int32, 16)

    @pl.when(c == 0)
    def _phase1():
        base = s * CHUNK
        @pl.when(s < NW - 1)
        def _full():
            pltpu.sync_copy(site_hbm.at[pl.ds(base, CHUNK)], site_v)
            pltpu.sync_copy(batch_hbm.at[pl.ds(base, CHUNK)],
                            idx_v.at[pl.ds(16, CHUNK)])

        @pl.when(s == NW - 1)
        def _tail():
            pltpu.sync_copy(site_hbm.at[pl.ds(base, LAST_CHUNK)],
                            site_v.at[pl.ds(0, LAST_CHUNK)])
            pltpu.sync_copy(batch_hbm.at[pl.ds(base, LAST_CHUNK)],
                            idx_v.at[pl.ds(16, LAST_CHUNK)])

        zeros = jnp.zeros((16,), jnp.float32)

        def zero_body(i, carry):
            acc_v[pl.ds(i * 16, 16)] = zeros
            return carry

        lax.fori_loop(0, N_SEG // 16, zero_body, 0)

        def one_vec(v):
            e = site_v[pl.ds(v * 16, 16)]
            idx = idx_v[pl.ds(16 + v * 16, 16)]
            idx_prev = idx_v[pl.ds(15 + v * 16, 16)]
            idx_next = idx_v[pl.ds(17 + v * 16, 16)]
            is_first = (lane == 0) | (idx != idx_prev)
            is_last = (lane == 15) | (idx != idx_next)
            csum = plsc.cumsum(e)
            plsc.addupdate_scatter(acc_v, [idx], csum, mask=is_last)
            plsc.addupdate_scatter(acc_v, [idx], e - csum, mask=is_first)

        def vec_body2(v, carry):
            one_vec(2 * v)
            one_vec(2 * v + 1)
            return carry

        nvec = jnp.where(s == NW - 1, LAST_NVEC, NVEC)
        lax.fori_loop(0, nvec // 2, vec_body2, 0)
        one_vec(nvec - 1)
        pltpu.sync_copy(acc_v, shared.at[s])

    plsc.subcore_barrier()

    @pl.when((c == 0) & (s < N_SEG // SEG_PER_W))
    def _phase2():
        col = s * SEG_PER_W
        pltpu.sync_copy(shared.at[:, pl.ds(col, SEG_PER_W)], red_v)
        for j in range(SEG_PER_W // 16):
            tot = jnp.zeros((16,), jnp.float32)
            for r in range(NW):
                tot = tot + red_v[r, pl.ds(j * 16, 16)]
            out_stage_v[pl.ds(j * 16, 16)] = tot
        pltpu.sync_copy(out_stage_v, out_hbm.at[pl.ds(col, SEG_PER_W)])


@functools.cache
def _make_seg_kernel():
  return pl.kernel(
    _seg_body,
    out_type=jax.ShapeDtypeStruct((N_SEG,), jnp.float32),
    mesh=plsc.VectorSubcoreMesh(core_axis_name="c", subcore_axis_name="s"),
    compiler_params=pltpu.CompilerParams(needs_layout_passes=False),
    scratch_types=[
        pltpu.VMEM((CHUNK,), jnp.float32),
        pltpu.VMEM((CHUNK + 32,), jnp.int32),
        pltpu.VMEM((N_SEG,), jnp.float32),
        pltpu.VMEM((NW, SEG_PER_W), jnp.float32),
        pltpu.VMEM((SEG_PER_W,), jnp.float32),
        pltpu.VMEM_SHARED((NW, N_SEG), jnp.float32),
    ],
  )


def kernel(coordinate, node_feat, batch, W1, b1, W2, b2):
    site = _site_energy(coordinate, node_feat, W1, b1, W2, b2)
    energy = _make_seg_kernel()(site, batch.astype(jnp.int32))
    return site, energy
